# baseline (speedup 1.0000x reference)
import functools

import jax
import jax.numpy as jnp
from jax import lax
from jax.experimental import pallas as pl
from jax.experimental.pallas import tpu as pltpu
from jax.experimental.pallas import tpu_sc as plsc

N = 16384
D = 128
DAY_ROWS = 32
MONTH_ROWS = 13
TABLE_ROWS = MONTH_ROWS * DAY_ROWS

NC = 2
NS = 16
L = 16
NW = NC * NS
ROWS_PER_W = N // NW
CHUNK = 64
NCHUNK = ROWS_PER_W // CHUNK


def _prep_body(xm_ref, xd_ref, day_ref, month_ref, table_ref, comb_ref):
    table_ref[...] = month_ref[...][:, None, :] + day_ref[...][None, :, :]
    di = (xd_ref[...] * 31.0).astype(jnp.int32)
    mi = (xm_ref[...] * 12.0).astype(jnp.int32)
    comb_ref[...] = mi * DAY_ROWS + di


def _prep(xm, xd, day_W, month_W):
    table, comb = pl.pallas_call(
        _prep_body,
        out_shape=(
            jax.ShapeDtypeStruct((MONTH_ROWS, DAY_ROWS, D), jnp.float32),
            jax.ShapeDtypeStruct((N,), jnp.int32),
        ),
    )(xm, xd, day_W, month_W)
    return table.reshape(TABLE_ROWS, D), comb


_mesh = plsc.VectorSubcoreMesh(
    core_axis_name="c", subcore_axis_name="s", num_cores=NC, num_subcores=NS
)


@functools.partial(
    pl.kernel,
    out_type=jax.ShapeDtypeStruct((N, D), jnp.float32),
    mesh=_mesh,
    compiler_params=pltpu.CompilerParams(skip_device_barrier=True),
    scratch_types=[
        pltpu.VMEM((ROWS_PER_W,), jnp.int32),
        pltpu.VMEM((NCHUNK, CHUNK, D), jnp.float32),
        pltpu.VMEM_SHARED((TABLE_ROWS, D), jnp.float32),
        pltpu.SemaphoreType.DMA,
        pltpu.SemaphoreType.DMA,
    ],
)
def _sc_lookup(comb_hbm, table_hbm, out_hbm, idx_v, rows_v, table_sh, gsem, wsem):
    sid = lax.axis_index("s")
    wid = sid * NC + lax.axis_index("c")
    base = wid * ROWS_PER_W

    @pl.when(sid == 0)
    def _():
        pltpu.sync_copy(table_hbm, table_sh)

    pltpu.sync_copy(comb_hbm.at[pl.ds(base, ROWS_PER_W)], idx_v)

    plsc.subcore_barrier()

    gathers = [
        pltpu.async_copy(
            table_sh.at[idx_v.at[pl.ds(c * CHUNK, CHUNK)]], rows_v.at[c], gsem
        )
        for c in range(NCHUNK)
    ]
    writes = []
    for c in range(NCHUNK):
        gathers[c].wait()
        writes.append(
            pltpu.async_copy(
                rows_v.at[c], out_hbm.at[pl.ds(base + c * CHUNK, CHUNK)], wsem
            )
        )
    for w in writes:
        w.wait()


def kernel(x, day_W, month_W):
    table, comb = _prep(x[:, 0], x[:, 1], day_W, month_W)
    return _sc_lookup(comb, table)


# speedup vs baseline: 1.0010x; 1.0010x over previous
import functools

import jax
import jax.numpy as jnp
from jax import lax
from jax.experimental import pallas as pl
from jax.experimental.pallas import tpu as pltpu
from jax.experimental.pallas import tpu_sc as plsc

N = 16384
D = 128
DAY_ROWS = 32
MONTH_ROWS = 13
TABLE_ROWS = MONTH_ROWS * DAY_ROWS

NC = 2
NS = 16
L = 16
NW = NC * NS
ROWS_PER_W = N // NW
CHUNK = 64
NCHUNK = ROWS_PER_W // CHUNK


def _prep_body(xm_ref, xd_ref, day_ref, month_ref, table_ref, comb_ref):
    table_ref[...] = month_ref[...][:, None, :] + day_ref[...][None, :, :]
    di = (xd_ref[...] * 31.0).astype(jnp.int32)
    mi = (xm_ref[...] * 12.0).astype(jnp.int32)
    comb_ref[...] = mi * DAY_ROWS + di


def _prep(xm, xd, day_W, month_W):
    table, comb = pl.pallas_call(
        _prep_body,
        out_shape=(
            jax.ShapeDtypeStruct((MONTH_ROWS, DAY_ROWS, D), jnp.float32),
            jax.ShapeDtypeStruct((N,), jnp.int32),
        ),
    )(xm, xd, day_W, month_W)
    return table.reshape(TABLE_ROWS, D), comb


_mesh = plsc.VectorSubcoreMesh(
    core_axis_name="c", subcore_axis_name="s", num_cores=NC, num_subcores=NS
)


@functools.partial(
    pl.kernel,
    out_type=jax.ShapeDtypeStruct((N, D), jnp.float32),
    mesh=_mesh,
    scratch_types=[
        pltpu.VMEM((ROWS_PER_W,), jnp.int32),
        pltpu.VMEM((NCHUNK, CHUNK, D), jnp.float32),
        pltpu.VMEM_SHARED((TABLE_ROWS, D), jnp.float32),
        pltpu.SemaphoreType.DMA,
        pltpu.SemaphoreType.DMA,
    ],
)
def _sc_lookup(comb_hbm, table_hbm, out_hbm, idx_v, rows_v, table_sh, gsem, wsem):
    sid = lax.axis_index("s")
    wid = sid * NC + lax.axis_index("c")
    base = wid * ROWS_PER_W

    @pl.when(sid == 0)
    def _():
        pltpu.sync_copy(table_hbm, table_sh)

    pltpu.sync_copy(comb_hbm.at[pl.ds(base, ROWS_PER_W)], idx_v)

    plsc.subcore_barrier()

    gathers = [
        pltpu.async_copy(
            table_sh.at[idx_v.at[pl.ds(c * CHUNK, CHUNK)]], rows_v.at[c], gsem
        )
        for c in range(NCHUNK)
    ]
    writes = []
    for c in range(NCHUNK):
        gathers[c].wait()
        writes.append(
            pltpu.async_copy(
                rows_v.at[c], out_hbm.at[pl.ds(base + c * CHUNK, CHUNK)], wsem
            )
        )
    for w in writes:
        w.wait()


def kernel(x, day_W, month_W):
    table, comb = _prep(x[:, 0], x[:, 1], day_W, month_W)
    return _sc_lookup(comb, table)


# speedup vs baseline: 1.0039x; 1.0029x over previous
import functools

import jax
import jax.numpy as jnp
from jax import lax
from jax.experimental import pallas as pl
from jax.experimental.pallas import tpu as pltpu
from jax.experimental.pallas import tpu_sc as plsc

N = 16384
D = 128
DAY_ROWS = 32
MONTH_ROWS = 13
TABLE_ROWS = MONTH_ROWS * DAY_ROWS

NC = 2
NS = 16
L = 16
NW = NC * NS
ROWS_PER_W = N // NW
CHUNK = 64
NCHUNK = ROWS_PER_W // CHUNK


def _prep_body(xm_ref, xd_ref, day_ref, month_ref, table_ref, comb_ref):
    table_ref[...] = month_ref[...][:, None, :] + day_ref[...][None, :, :]
    di = (xd_ref[...] * 31.0).astype(jnp.int32)
    mi = (xm_ref[...] * 12.0).astype(jnp.int32)
    comb_ref[...] = mi * DAY_ROWS + di


def _prep(xm, xd, day_W, month_W):
    table, comb = pl.pallas_call(
        _prep_body,
        out_shape=(
            jax.ShapeDtypeStruct((MONTH_ROWS, DAY_ROWS, D), jnp.float32),
            jax.ShapeDtypeStruct((N,), jnp.int32),
        ),
        compiler_params=pltpu.CompilerParams(
            allow_input_fusion=[True, True, False, False]
        ),
    )(xm, xd, day_W, month_W)
    return table.reshape(TABLE_ROWS, D), comb


_mesh = plsc.VectorSubcoreMesh(
    core_axis_name="c", subcore_axis_name="s", num_cores=NC, num_subcores=NS
)


@functools.partial(
    pl.kernel,
    out_type=jax.ShapeDtypeStruct((N, D), jnp.float32),
    mesh=_mesh,
    scratch_types=[
        pltpu.VMEM((ROWS_PER_W,), jnp.int32),
        pltpu.VMEM((NCHUNK, CHUNK, D), jnp.float32),
        pltpu.VMEM_SHARED((TABLE_ROWS, D), jnp.float32),
        pltpu.SemaphoreType.DMA,
        pltpu.SemaphoreType.DMA,
    ],
)
def _sc_lookup(comb_hbm, table_hbm, out_hbm, idx_v, rows_v, table_sh, gsem, wsem):
    sid = lax.axis_index("s")
    wid = sid * NC + lax.axis_index("c")
    base = wid * ROWS_PER_W

    @pl.when(sid == 0)
    def _():
        pltpu.sync_copy(table_hbm, table_sh)

    pltpu.sync_copy(comb_hbm.at[pl.ds(base, ROWS_PER_W)], idx_v)

    plsc.subcore_barrier()

    gathers = [
        pltpu.async_copy(
            table_sh.at[idx_v.at[pl.ds(c * CHUNK, CHUNK)]], rows_v.at[c], gsem
        )
        for c in range(NCHUNK)
    ]
    writes = []
    for c in range(NCHUNK):
        gathers[c].wait()
        writes.append(
            pltpu.async_copy(
                rows_v.at[c], out_hbm.at[pl.ds(base + c * CHUNK, CHUNK)], wsem
            )
        )
    for w in writes:
        w.wait()


def kernel(x, day_W, month_W):
    table, comb = _prep(x[:, 0], x[:, 1], day_W, month_W)
    return _sc_lookup(comb, table)
